# baseline (speedup 1.0000x reference)
import functools

import jax
import jax.numpy as jnp
from jax.experimental import pallas as pl
from jax.experimental.pallas import tpu as pltpu
from jax.experimental.pallas import tpu_sc as plsc

_BB = 4


def _make_occ_sc(L):
    info = plsc.get_sparse_core_info()
    ns = info.num_subcores
    ppw = L // ns

    mesh = plsc.VectorSubcoreMesh(core_axis_name="c", subcore_axis_name="s")

    @functools.partial(
        pl.kernel, mesh=mesh,
        out_type=jax.ShapeDtypeStruct((256,), jnp.float32),
        scratch_types=[
            pltpu.VMEM((2, ppw), jnp.int32),
            pltpu.VMEM((256,), jnp.float32),
            pltpu.VMEM((256,), jnp.float32),
            pltpu.VMEM_SHARED((ns, 256), jnp.float32),
        ],
        compiler_params=pltpu.CompilerParams(needs_layout_passes=False),
    )
    def occ_sc(mask_hbm, out_hbm, pairs_v, table_v, read_v, shared_sp):
        cid = jax.lax.axis_index("c")
        sid = jax.lax.axis_index("s")

        @pl.when(cid == 0)
        def _work():
            pltpu.sync_copy(mask_hbm.at[:, pl.ds(sid * ppw, ppw)], pairs_v)
            zeros16 = jnp.zeros((16,), jnp.float32)
            for i in range(16):
                table_v[pl.ds(i * 16, 16)] = zeros16
            ones16 = jnp.full((16,), 1.0, jnp.float32)
            for i in range(ppw // 16):
                bvec = pairs_v[0, pl.ds(i * 16, 16)]
                rvec = pairs_v[1, pl.ds(i * 16, 16)]
                plsc.store_scatter(table_v, [bvec * 16 + rvec], ones16)
            pltpu.sync_copy(table_v, shared_sp.at[sid])
            plsc.subcore_barrier()

            @pl.when(sid == 0)
            def _combine():
                for t in range(1, ns):
                    pltpu.sync_copy(shared_sp.at[t], read_v)
                    for i in range(16):
                        sl = pl.ds(i * 16, 16)
                        table_v[sl] = jnp.maximum(table_v[sl], read_v[sl])
                pltpu.sync_copy(table_v, out_hbm)

    return occ_sc


def _dense_body(x_ref, occ_ref, w_ref, u_ref, out_ref, g_ref):
    b = pl.program_id(0)
    H = w_ref.shape[0]
    wb = jnp.broadcast_to(w_ref[...], (H, H))

    for bb in range(_BB):
        x_blk = x_ref[bb, 16:, :]
        gpre = jnp.dot(x_blk, wb, preferred_element_type=jnp.float32)
        g = jax.nn.sigmoid(gpre)
        out_ref[bb, 16:, :] = x_blk * g
        g_ref[bb, 16:, :] = g[:, 0:1]

        m16 = occ_ref[pl.ds((b * _BB + bb) * 16, 16), :]
        rv = x_ref[bb, 0:1, :]
        a_b = jnp.sum(jnp.dot(rv, u_ref[...],
                              preferred_element_type=jnp.float32))
        x16 = x_ref[bb, 0:16, :]
        gpre16 = jnp.dot(x16, wb, preferred_element_type=jnp.float32)
        g16 = jax.nn.sigmoid(gpre16 + m16 * a_b)
        out_ref[bb, 0:16, :] = x16 * g16 + (m16 * (1.0 - g16)) * rv
        g_ref[bb, 0:16, :] = g16[:, 0:1]


def kernel(x, mask_nonzero, w, u):
    B, N, H = x.shape
    L = mask_nonzero.shape[1]
    occ = _make_occ_sc(L)(mask_nonzero).reshape(256, 1)

    out, g = pl.pallas_call(
        _dense_body,
        grid=(B // _BB,),
        in_specs=[
            pl.BlockSpec((_BB, N, H), lambda b: (b, 0, 0)),
            pl.BlockSpec((256, 1), lambda b: (0, 0)),
            pl.BlockSpec((H, 1), lambda b: (0, 0)),
            pl.BlockSpec((H, 1), lambda b: (0, 0)),
        ],
        out_specs=[
            pl.BlockSpec((_BB, N, H), lambda b: (b, 0, 0)),
            pl.BlockSpec((_BB, N, 1), lambda b: (b, 0, 0)),
        ],
        out_shape=[
            jax.ShapeDtypeStruct((B, N, H), x.dtype),
            jax.ShapeDtypeStruct((B, N, 1), x.dtype),
        ],
        compiler_params=pltpu.CompilerParams(
            dimension_semantics=("parallel",)),
    )(x, occ, w, u)
    return out, g


# speedup vs baseline: 1.0884x; 1.0884x over previous
import functools

import jax
import jax.numpy as jnp
from jax.experimental import pallas as pl
from jax.experimental.pallas import tpu as pltpu
from jax.experimental.pallas import tpu_sc as plsc

_BB = 4


def _make_occ_sc(L):
    info = plsc.get_sparse_core_info()
    ns = info.num_subcores
    ppw = L // ns

    mesh = plsc.VectorSubcoreMesh(core_axis_name="c", subcore_axis_name="s")

    @functools.partial(
        pl.kernel, mesh=mesh,
        out_type=jax.ShapeDtypeStruct((256,), jnp.float32),
        scratch_types=[
            pltpu.VMEM((2, ppw), jnp.int32),
            pltpu.VMEM((256,), jnp.float32),
            pltpu.VMEM((256,), jnp.float32),
            pltpu.VMEM_SHARED((ns, 256), jnp.float32),
        ],
        compiler_params=pltpu.CompilerParams(needs_layout_passes=False),
    )
    def occ_sc(mask_hbm, out_hbm, pairs_v, table_v, read_v, shared_sp):
        cid = jax.lax.axis_index("c")
        sid = jax.lax.axis_index("s")

        @pl.when(cid == 0)
        def _work():
            pltpu.sync_copy(mask_hbm.at[:, pl.ds(sid * ppw, ppw)], pairs_v)
            zeros16 = jnp.zeros((16,), jnp.float32)
            for i in range(16):
                table_v[pl.ds(i * 16, 16)] = zeros16
            ones16 = jnp.full((16,), 1.0, jnp.float32)
            for i in range(ppw // 16):
                bvec = pairs_v[0, pl.ds(i * 16, 16)]
                rvec = pairs_v[1, pl.ds(i * 16, 16)]
                plsc.store_scatter(table_v, [bvec * 16 + rvec], ones16)
            pltpu.sync_copy(table_v, shared_sp.at[sid])
            plsc.subcore_barrier()

            @pl.when(sid == 0)
            def _combine():
                for t in range(1, ns):
                    pltpu.sync_copy(shared_sp.at[t], read_v)
                    for i in range(16):
                        sl = pl.ds(i * 16, 16)
                        table_v[sl] = jnp.maximum(table_v[sl], read_v[sl])
                pltpu.sync_copy(table_v, out_hbm)

    return occ_sc


def _bulk_body(x_ref, w_ref, out_ref, g_ref):
    H = w_ref.shape[0]
    wb = jnp.broadcast_to(w_ref[...], (H, H))
    for bb in range(_BB):
        x_blk = x_ref[bb]
        gpre = jnp.dot(x_blk, wb, preferred_element_type=jnp.float32)
        g = jax.nn.sigmoid(gpre)
        out_ref[bb] = x_blk * g
        g_ref[bb] = g[:, 0:1]


def _patch_body(x_ref, occ_ref, w_ref, u_ref, out0_ref, g0_ref,
                out_ref, g_ref):
    del out0_ref, g0_ref
    B = x_ref.shape[0]
    H = w_ref.shape[0]
    wb = jnp.broadcast_to(w_ref[...], (H, H))
    x16 = x_ref[...].reshape(B * 16, H)
    roots = x_ref[:, 0, :]
    root_mat = jnp.broadcast_to(roots[:, None, :], (B, 16, H)
                                ).reshape(B * 16, H)
    a = jnp.dot(roots, u_ref[...], preferred_element_type=jnp.float32)
    a_rep = jnp.broadcast_to(a[:, None, :], (B, 16, 1)).reshape(B * 16, 1)
    m = occ_ref[...]
    gpre = jnp.dot(x16, wb, preferred_element_type=jnp.float32) + m * a_rep
    g = jax.nn.sigmoid(gpre)
    res = x16 * g + (m * (1.0 - g)) * root_mat
    out_ref[...] = res.reshape(B, 16, H)
    g_ref[...] = g[:, 0:1].reshape(B, 16, 1)


def kernel(x, mask_nonzero, w, u):
    B, N, H = x.shape
    L = mask_nonzero.shape[1]
    occ = _make_occ_sc(L)(mask_nonzero).reshape(256, 1)

    out0, g0 = pl.pallas_call(
        _bulk_body,
        grid=(B // _BB,),
        in_specs=[
            pl.BlockSpec((_BB, N, H), lambda b: (b, 0, 0)),
            pl.BlockSpec((H, 1), lambda b: (0, 0)),
        ],
        out_specs=[
            pl.BlockSpec((_BB, N, H), lambda b: (b, 0, 0)),
            pl.BlockSpec((_BB, N, 1), lambda b: (b, 0, 0)),
        ],
        out_shape=[
            jax.ShapeDtypeStruct((B, N, H), x.dtype),
            jax.ShapeDtypeStruct((B, N, 1), x.dtype),
        ],
        compiler_params=pltpu.CompilerParams(
            dimension_semantics=("parallel",)),
    )(x, w)

    out, g = pl.pallas_call(
        _patch_body,
        grid=(1,),
        in_specs=[
            pl.BlockSpec((B, 16, H), lambda i: (0, 0, 0)),
            pl.BlockSpec((256, 1), lambda i: (0, 0)),
            pl.BlockSpec((H, 1), lambda i: (0, 0)),
            pl.BlockSpec((H, 1), lambda i: (0, 0)),
            pl.BlockSpec((B, 16, H), lambda i: (0, 0, 0)),
            pl.BlockSpec((B, 16, 1), lambda i: (0, 0, 0)),
        ],
        out_specs=[
            pl.BlockSpec((B, 16, H), lambda i: (0, 0, 0)),
            pl.BlockSpec((B, 16, 1), lambda i: (0, 0, 0)),
        ],
        out_shape=[
            jax.ShapeDtypeStruct((B, N, H), x.dtype),
            jax.ShapeDtypeStruct((B, N, 1), x.dtype),
        ],
        input_output_aliases={4: 0, 5: 1},
    )(x, occ, w, u, out0, g0)
    return out, g


# speedup vs baseline: 1.4201x; 1.3047x over previous
import jax
import jax.numpy as jnp
from jax.experimental import pallas as pl
from jax.experimental.pallas import tpu as pltpu

_BB = 4


def _build_occ(m_ref, occ_ref):
    bc = m_ref[0]
    rc = m_ref[1]
    ids = bc * 16 + rc
    widx = jax.lax.shift_right_logical(ids, 5)
    bit = jax.lax.shift_left(jnp.int32(1), ids & 31)

    words = []
    for k in range(8):
        sel = jnp.where(widx == k, bit, 0)
        while sel.shape[0] > 1:
            h = sel.shape[0] // 2
            sel = sel[:h] | sel[h:]
        words.append(sel)
    w8 = jnp.concatenate(words, axis=0)

    wrep = jnp.broadcast_to(w8[:, None, :], (8, 32, 128)).reshape(256, 128)
    t = jax.lax.broadcasted_iota(jnp.int32, (256, 128), 0) & 31
    bits = jax.lax.shift_right_logical(wrep, t) & 1
    occ_ref[...] = jnp.max(bits, axis=1, keepdims=True).astype(jnp.float32)


def _dense_body(x_ref, m_ref, w_ref, u_ref, out_ref, g_ref, occ_ref):
    b = pl.program_id(0)
    H = w_ref.shape[0]

    @pl.when(b == 0)
    def _occ():
        _build_occ(m_ref, occ_ref)

    wb = jnp.broadcast_to(w_ref[...], (H, H))

    for bb in range(_BB):
        x_blk = x_ref[bb, 16:, :]
        gpre = jnp.dot(x_blk, wb, preferred_element_type=jnp.float32)
        g = jax.nn.sigmoid(gpre)
        out_ref[bb, 16:, :] = x_blk * g
        g_ref[bb, 16:, :] = g[:, 0:1]

        m16 = occ_ref[pl.ds((b * _BB + bb) * 16, 16), :]
        rv = x_ref[bb, 0:1, :]
        a_b = jnp.sum(jnp.dot(rv, u_ref[...],
                              preferred_element_type=jnp.float32))
        x16 = x_ref[bb, 0:16, :]
        gpre16 = jnp.dot(x16, wb, preferred_element_type=jnp.float32)
        g16 = jax.nn.sigmoid(gpre16 + m16 * a_b)
        out_ref[bb, 0:16, :] = x16 * g16 + (m16 * (1.0 - g16)) * rv
        g_ref[bb, 0:16, :] = g16[:, 0:1]


def kernel(x, mask_nonzero, w, u):
    B, N, H = x.shape
    L = mask_nonzero.shape[1]
    m3 = mask_nonzero.reshape(2, L // 128, 128)

    out, g = pl.pallas_call(
        _dense_body,
        grid=(B // _BB,),
        in_specs=[
            pl.BlockSpec((_BB, N, H), lambda b: (b, 0, 0)),
            pl.BlockSpec((2, L // 128, 128), lambda b: (0, 0, 0)),
            pl.BlockSpec((H, 1), lambda b: (0, 0)),
            pl.BlockSpec((H, 1), lambda b: (0, 0)),
        ],
        out_specs=[
            pl.BlockSpec((_BB, N, H), lambda b: (b, 0, 0)),
            pl.BlockSpec((_BB, N, 1), lambda b: (b, 0, 0)),
        ],
        out_shape=[
            jax.ShapeDtypeStruct((B, N, H), x.dtype),
            jax.ShapeDtypeStruct((B, N, 1), x.dtype),
        ],
        scratch_shapes=[
            pltpu.VMEM((256, 1), jnp.float32),
        ],
        compiler_params=pltpu.CompilerParams(
            dimension_semantics=("arbitrary",)),
    )(x, m3, w, u)
    return out, g
